# Initial kernel scaffold; baseline (speedup 1.0000x reference)
import jax
import jax.numpy as jnp
from jax.experimental import pallas as pl


def kernel(E, edge_index0, eids0, edge_index1, eids1, Wrel0, Wroot0, b0, Wrel1, Wroot1, b1, W1, bs1, W2, ifdropout):
    raise NotImplementedError("write your pallas kernel here")


# speedup vs baseline: 7.9700x; 7.9700x over previous
import functools

import jax
import jax.numpy as jnp
from jax import lax
from jax.experimental import pallas as pl
from jax.experimental.pallas import tpu as pltpu
from jax.experimental.pallas import tpu_sc as plsc

N_TOTAL = 50000
N_SUB = 10000
N_EDGES = 320000
D = 128
HIDDEN = 128
NREG = 10000

NPAD = 10240
NTILE = 16
ROWS_W = NPAD // NTILE
GCH = ROWS_W // 128
SBCH = 16
CHW = 64
NBLK = 20
EPAD = NTILE * NBLK * SBCH * CHW

_sc_mesh = plsc.VectorSubcoreMesh(core_axis_name="c", subcore_axis_name="s")


def _gather_body(table, idx, out, idx_v, buf0, buf1, sem0, sem1):
    c = lax.axis_index("c")
    s = lax.axis_index("s")
    bufs = (buf0, buf1)
    sems = (sem0, sem1)
    pltpu.sync_copy(idx.at[c, s], idx_v)
    pltpu.async_copy(table.at[idx_v.at[0]], buf0, sem0)
    for j in range(GCH):
        pltpu.make_async_copy(table.at[idx_v.at[j]], bufs[j % 2],
                              sems[j % 2]).wait()
        if j + 1 < GCH:
            pltpu.async_copy(table.at[idx_v.at[j + 1]], bufs[(j + 1) % 2],
                             sems[(j + 1) % 2])
        pltpu.sync_copy(bufs[j % 2],
                        out.at[c, pl.ds(s * ROWS_W + j * 128, 128)])


_gather_call = functools.partial(
    pl.kernel,
    _gather_body,
    out_type=jax.ShapeDtypeStruct((2, NPAD, D), jnp.float32),
    mesh=_sc_mesh,
    scratch_types=[
        pltpu.VMEM((GCH, 128), jnp.int32),
        pltpu.VMEM((128, D), jnp.float32),
        pltpu.VMEM((128, D), jnp.float32),
        pltpu.SemaphoreType.DMA,
        pltpu.SemaphoreType.DMA,
    ],
)


def _mm_body(x_ref, wrel_ref, wroot_ref, b_ref, xw_ref, xr_ref):
    x = x_ref[0]
    xw_ref[0] = jnp.dot(x, wrel_ref[0], preferred_element_type=jnp.float32)
    xr_ref[0] = (
        jnp.dot(x, wroot_ref[0], preferred_element_type=jnp.float32)
        + b_ref[0]
    )


def _mm_call(x, wrel, wroot, bstack):
    bm = 2048
    return pl.pallas_call(
        _mm_body,
        grid=(2, NPAD // bm),
        in_specs=[
            pl.BlockSpec((1, bm, D), lambda p, i: (p, i, 0)),
            pl.BlockSpec((1, D, D), lambda p, i: (p, 0, 0)),
            pl.BlockSpec((1, D, D), lambda p, i: (p, 0, 0)),
            pl.BlockSpec((1, 1, D), lambda p, i: (p, 0, 0)),
        ],
        out_specs=[
            pl.BlockSpec((1, bm, D), lambda p, i: (p, i, 0)),
            pl.BlockSpec((1, bm, D), lambda p, i: (p, i, 0)),
        ],
        out_shape=[
            jax.ShapeDtypeStruct((2, NPAD, D), jnp.float32),
            jax.ShapeDtypeStruct((2, NPAD, D), jnp.float32),
        ],
    )(x, wrel, wroot, bstack.reshape(2, 1, D))


def _edge_body(xw, srcp, dstp, agg_out, deg_out,
               agg_sh, deg_sh, src_a, dst_a, src_b, dst_b,
               rows0, rows1, zbuf, zvec, ones_v,
               semg0, semg1, semi_s, semi_d, semsc0, semsc1):
    c = lax.axis_index("c")
    s = lax.axis_index("s")

    def _zb(i, _):
        for k in range(8):
            zbuf[i, pl.ds(k * 16, 16)] = jnp.zeros((16,), jnp.float32)
        return _
    lax.fori_loop(0, 16, _zb, None)

    def _zv(i, _):
        zvec[pl.ds(i * 16, 16)] = jnp.zeros((16,), jnp.float32)
        return _
    lax.fori_loop(0, ROWS_W // 16, _zv, None)

    def _ov(i, _):
        ones_v[pl.ds(i * 16, 16)] = jnp.ones((16,), jnp.float32)
        return _
    lax.fori_loop(0, CHW // 16, _ov, None)

    for t in range(ROWS_W // 16):
        pltpu.sync_copy(zbuf, agg_sh.at[pl.ds(s * ROWS_W + t * 16, 16)])
    pltpu.sync_copy(zvec, deg_sh.at[pl.ds(s * ROWS_W, ROWS_W)])
    plsc.subcore_barrier()

    rows = (rows0, rows1)
    semg = (semg0, semg1)
    semsc = (semsc0, semsc1)

    pltpu.sync_copy(srcp.at[c, s, 0], src_a)
    pltpu.sync_copy(dstp.at[c, s, 0], dst_a)
    pltpu.async_copy(xw.at[src_a.at[0]], rows0, semg0)
    pltpu.async_copy(srcp.at[c, s, 1], src_b, semi_s)
    pltpu.async_copy(dstp.at[c, s, 1], dst_b, semi_d)

    def _pair(bb, _):
        for half in range(2):
            b = 2 * bb + half
            sv, dv = (src_a, dst_a) if half == 0 else (src_b, dst_b)
            nsv, ndv = (src_b, dst_b) if half == 0 else (src_a, dst_a)
            for j in range(SBCH):
                pltpu.make_async_copy(
                    xw.at[sv.at[j]], rows[j % 2], semg[j % 2]).wait()
                if j >= 1:
                    pltpu.make_async_copy(
                        rows[(j - 1) % 2], agg_sh.at[dv.at[j - 1]],
                        semsc[(j - 1) % 2]).wait()
                    pltpu.make_async_copy(
                        ones_v, deg_sh.at[dv.at[j - 1]],
                        semsc[(j - 1) % 2]).wait()
                if j + 1 < SBCH:
                    pltpu.async_copy(
                        xw.at[sv.at[j + 1]], rows[(j + 1) % 2],
                        semg[(j + 1) % 2])
                else:
                    @pl.when(b < NBLK - 1)
                    def _():
                        pltpu.make_async_copy(
                            srcp.at[c, s, b + 1], nsv, semi_s).wait()
                        pltpu.make_async_copy(
                            dstp.at[c, s, b + 1], ndv, semi_d).wait()
                        pltpu.async_copy(xw.at[nsv.at[0]], rows0, semg0)
                pltpu.async_copy(rows[j % 2], agg_sh.at[dv.at[j]],
                                 semsc[j % 2], add=True)
                pltpu.async_copy(ones_v, deg_sh.at[dv.at[j]],
                                 semsc[j % 2], add=True)

            pltpu.make_async_copy(
                rows[(SBCH - 1) % 2], agg_sh.at[dv.at[SBCH - 1]],
                semsc[(SBCH - 1) % 2]).wait()
            pltpu.make_async_copy(
                ones_v, deg_sh.at[dv.at[SBCH - 1]],
                semsc[(SBCH - 1) % 2]).wait()

            @pl.when(b < NBLK - 2)
            def _():
                pltpu.async_copy(srcp.at[c, s, b + 2], sv, semi_s)
                pltpu.async_copy(dstp.at[c, s, b + 2], dv, semi_d)
        return _
    lax.fori_loop(0, NBLK // 2, _pair, None)

    plsc.subcore_barrier()

    for t in range(ROWS_W // CHW):
        pltpu.sync_copy(agg_sh.at[pl.ds(s * ROWS_W + t * CHW, CHW)], rows0)
        pltpu.sync_copy(rows0, agg_out.at[c, pl.ds(s * ROWS_W + t * CHW, CHW)])
    pltpu.sync_copy(deg_sh.at[pl.ds(s * ROWS_W, ROWS_W)], zvec)
    pltpu.sync_copy(zvec, deg_out.at[c, pl.ds(s * ROWS_W, ROWS_W)])


_edge_call = functools.partial(
    pl.kernel,
    _edge_body,
    out_type=(
        jax.ShapeDtypeStruct((2, NPAD, D), jnp.float32),
        jax.ShapeDtypeStruct((2, NPAD), jnp.float32),
    ),
    mesh=_sc_mesh,
    scratch_types=[
        pltpu.VMEM_SHARED((NPAD, D), jnp.float32),
        pltpu.VMEM_SHARED((NPAD,), jnp.float32),
        pltpu.VMEM((SBCH, CHW), jnp.int32),
        pltpu.VMEM((SBCH, CHW), jnp.int32),
        pltpu.VMEM((SBCH, CHW), jnp.int32),
        pltpu.VMEM((SBCH, CHW), jnp.int32),
        pltpu.VMEM((CHW, D), jnp.float32),
        pltpu.VMEM((CHW, D), jnp.float32),
        pltpu.VMEM((16, D), jnp.float32),
        pltpu.VMEM((ROWS_W,), jnp.float32),
        pltpu.VMEM((CHW,), jnp.float32),
        pltpu.SemaphoreType.DMA,
        pltpu.SemaphoreType.DMA,
        pltpu.SemaphoreType.DMA,
        pltpu.SemaphoreType.DMA,
        pltpu.SemaphoreType.DMA,
        pltpu.SemaphoreType.DMA,
    ],
)


def _post_body(agg_ref, deg_ref, xr_ref, w1_ref, bs1_ref, w2_ref,
               o_ref, h_sc, s_sc):
    t = pl.program_id(0)

    @pl.when(t < 20)
    def _pass1():
        i = t % 10
        deg = jnp.maximum(jnp.transpose(deg_ref[0], (1, 0)), 1.0)
        h = jnp.maximum(agg_ref[0] / deg + xr_ref[0], 0.0)
        tt = jnp.tanh(
            lax.dot_general(h, w1_ref[...], (((1,), (1,)), ((), ())),
                            preferred_element_type=jnp.float32)
            + bs1_ref[...][None, :]
        )
        rows = i * 1024 + lax.broadcasted_iota(jnp.int32, (1024, 1), 0)
        maskf = (rows < NREG).astype(jnp.float32)
        contrib = jnp.sum(tt * w2_ref[...][0][None, :] * maskf)
        base = pl.multiple_of(i * 1024, 1024)

        @pl.when(t < 10)
        def _():
            h_sc[0, pl.ds(base, 1024)] = h
            s_sc[0] = jnp.where(i == 0, 0.0, s_sc[0]) + contrib

        @pl.when(t >= 10)
        def _():
            h_sc[1, pl.ds(base, 1024)] = h
            s_sc[1] = jnp.where(i == 0, 0.0, s_sc[1]) + contrib

    @pl.when(t >= 20)
    def _pass2():
        i = t - 20
        base = pl.multiple_of(i * 1000, 1000)
        s0 = s_sc[0] / float(NREG)
        s1 = s_sc[1] / float(NREG)
        m = jnp.maximum(s0, s1)
        e0 = jnp.exp(s0 - m)
        e1 = jnp.exp(s1 - m)
        b0 = e0 / (e0 + e1)
        o_ref[...] = (b0 * h_sc[0, pl.ds(base, 1000)]
                      + (1.0 - b0) * h_sc[1, pl.ds(base, 1000)])


def _post_call(agg, deg, xr, w1, bs1, w2):
    return pl.pallas_call(
        _post_body,
        grid=(30,),
        in_specs=[
            pl.BlockSpec((1, 1024, D),
                         lambda t: (jnp.where(t < 20, t // 10, 0),
                                    jnp.where(t < 20, t % 10, 0), 0)),
            pl.BlockSpec((1, 1, 1024),
                         lambda t: (jnp.where(t < 20, t, 0), 0, 0)),
            pl.BlockSpec((1, 1024, D),
                         lambda t: (jnp.where(t < 20, t // 10, 0),
                                    jnp.where(t < 20, t % 10, 0), 0)),
            pl.BlockSpec((HIDDEN, D), lambda t: (0, 0)),
            pl.BlockSpec((HIDDEN,), lambda t: (0,)),
            pl.BlockSpec((1, HIDDEN), lambda t: (0, 0)),
        ],
        out_specs=pl.BlockSpec(
            (1000, D), lambda t: (jnp.where(t >= 20, t - 20, 0), 0)),
        out_shape=jax.ShapeDtypeStruct((NREG, D), jnp.float32),
        scratch_shapes=[pltpu.VMEM((2, NPAD, D), jnp.float32),
                        pltpu.SMEM((2,), jnp.float32)],
    )(agg, deg.reshape(2 * NPAD // 1024, 1, 1024), xr, w1, bs1, w2)


def kernel(E, edge_index0, eids0, edge_index1, eids1, Wrel0, Wroot0, b0,
           Wrel1, Wroot1, b1, W1, bs1, W2, ifdropout):
    del ifdropout
    pad_rows = (jnp.arange(NPAD - N_SUB, dtype=jnp.int32) * 199) % N_TOTAL
    eids = jnp.stack([eids0, eids1])
    eids_pad = jnp.concatenate(
        [eids, jnp.broadcast_to(pad_rows, (2, NPAD - N_SUB))], axis=1
    ).reshape(2, NTILE, GCH, 128)

    padn = EPAD - N_EDGES
    ar = jnp.arange(padn, dtype=jnp.int32)
    pad_dst = N_SUB + (ar % (NPAD - N_SUB))
    src_idx = jnp.concatenate(
        [edge_index0[0], pad_dst, edge_index1[0] + NPAD, NPAD + pad_dst]
    ).reshape(2, NTILE, NBLK, SBCH, CHW)
    dst_idx = jnp.concatenate(
        [edge_index0[1], pad_dst, edge_index1[1], pad_dst]
    ).reshape(2, NTILE, NBLK, SBCH, CHW)

    wrel = jnp.stack([Wrel0[0], Wrel1[0]])
    wroot = jnp.stack([Wroot0, Wroot1])
    bstack = jnp.stack([b0, b1])

    x = _gather_call()(E, eids_pad)
    xw, xr = _mm_call(x, wrel, wroot, bstack)
    agg, deg = _edge_call()(xw.reshape(2 * NPAD, D), src_idx, dst_idx)
    return _post_call(agg, deg, xr, W1, bs1, W2)


# speedup vs baseline: 10.5231x; 1.3203x over previous
import functools

import jax
import jax.numpy as jnp
from jax import lax
from jax.experimental import pallas as pl
from jax.experimental.pallas import tpu as pltpu
from jax.experimental.pallas import tpu_sc as plsc

N_TOTAL = 50000
N_SUB = 10000
N_EDGES = 320000
D = 128
HIDDEN = 128
NREG = 10000

NPAD = 10240
NTILE = 16
ROWS_W = NPAD // NTILE
GCH = ROWS_W // 128
SBCH = 8
CHW = 128
NBLK = 20
EPAD = NTILE * NBLK * SBCH * CHW

_sc_mesh = plsc.VectorSubcoreMesh(core_axis_name="c", subcore_axis_name="s")


def _gather_body(table, idx, out, idx_v, buf0, buf1, sem0, sem1):
    c = lax.axis_index("c")
    s = lax.axis_index("s")
    bufs = (buf0, buf1)
    sems = (sem0, sem1)
    pltpu.sync_copy(idx.at[c, s], idx_v)
    pltpu.async_copy(table.at[idx_v.at[0]], buf0, sem0)
    for j in range(GCH):
        pltpu.make_async_copy(table.at[idx_v.at[j]], bufs[j % 2],
                              sems[j % 2]).wait()
        if j + 1 < GCH:
            pltpu.async_copy(table.at[idx_v.at[j + 1]], bufs[(j + 1) % 2],
                             sems[(j + 1) % 2])
        pltpu.sync_copy(bufs[j % 2],
                        out.at[c, pl.ds(s * ROWS_W + j * 128, 128)])


_gather_call = functools.partial(
    pl.kernel,
    _gather_body,
    out_type=jax.ShapeDtypeStruct((2, NPAD, D), jnp.float32),
    mesh=_sc_mesh,
    scratch_types=[
        pltpu.VMEM((GCH, 128), jnp.int32),
        pltpu.VMEM((128, D), jnp.float32),
        pltpu.VMEM((128, D), jnp.float32),
        pltpu.SemaphoreType.DMA,
        pltpu.SemaphoreType.DMA,
    ],
)


def _mm_body(x_ref, wrel_ref, wroot_ref, b_ref, xw_ref, xr_ref):
    x = x_ref[0]
    xw_ref[0] = jnp.dot(x, wrel_ref[0], preferred_element_type=jnp.float32)
    xr_ref[0] = (
        jnp.dot(x, wroot_ref[0], preferred_element_type=jnp.float32)
        + b_ref[0]
    )


def _mm_call(x, wrel, wroot, bstack):
    bm = 2048
    return pl.pallas_call(
        _mm_body,
        grid=(2, NPAD // bm),
        in_specs=[
            pl.BlockSpec((1, bm, D), lambda p, i: (p, i, 0)),
            pl.BlockSpec((1, D, D), lambda p, i: (p, 0, 0)),
            pl.BlockSpec((1, D, D), lambda p, i: (p, 0, 0)),
            pl.BlockSpec((1, 1, D), lambda p, i: (p, 0, 0)),
        ],
        out_specs=[
            pl.BlockSpec((1, bm, D), lambda p, i: (p, i, 0)),
            pl.BlockSpec((1, bm, D), lambda p, i: (p, i, 0)),
        ],
        out_shape=[
            jax.ShapeDtypeStruct((2, NPAD, D), jnp.float32),
            jax.ShapeDtypeStruct((2, NPAD, D), jnp.float32),
        ],
    )(x, wrel, wroot, bstack.reshape(2, 1, D))


def _edge_body(xw, srcp, dstp, agg_out, deg_out,
               agg_sh, deg_sh, src_a, dst_a, src_b, dst_b,
               rows0, rows1, zvec, ones_v,
               semg0, semg1, semi_s, semi_d, semsc0, semsc1):
    c = lax.axis_index("c")
    s = lax.axis_index("s")

    def _zr(i, _):
        for k in range(8):
            rows0[i, pl.ds(k * 16, 16)] = jnp.zeros((16,), jnp.float32)
        return _
    lax.fori_loop(0, CHW, _zr, None)

    def _zv(i, _):
        zvec[pl.ds(i * 16, 16)] = jnp.zeros((16,), jnp.float32)
        return _
    lax.fori_loop(0, ROWS_W // 16, _zv, None)

    def _ov(i, _):
        ones_v[pl.ds(i * 16, 16)] = jnp.ones((16,), jnp.float32)
        return _
    lax.fori_loop(0, CHW // 16, _ov, None)

    for t in range(ROWS_W // CHW):
        pltpu.sync_copy(rows0, agg_sh.at[pl.ds(s * ROWS_W + t * CHW, CHW)])
    pltpu.sync_copy(zvec, deg_sh.at[pl.ds(s * ROWS_W, ROWS_W)])
    plsc.subcore_barrier()

    rows = (rows0, rows1)
    semg = (semg0, semg1)
    semsc = (semsc0, semsc1)

    pltpu.sync_copy(srcp.at[c, s, 0], src_a)
    pltpu.sync_copy(dstp.at[c, s, 0], dst_a)
    pltpu.async_copy(xw.at[src_a.at[0]], rows0, semg0)
    pltpu.async_copy(srcp.at[c, s, 1], src_b, semi_s)
    pltpu.async_copy(dstp.at[c, s, 1], dst_b, semi_d)

    def _pair(bb, _):
        for half in range(2):
            b = 2 * bb + half
            sv, dv = (src_a, dst_a) if half == 0 else (src_b, dst_b)
            nsv, ndv = (src_b, dst_b) if half == 0 else (src_a, dst_a)
            for j in range(SBCH):
                pltpu.make_async_copy(
                    xw.at[sv.at[j]], rows[j % 2], semg[j % 2]).wait()
                if j >= 1:
                    pltpu.make_async_copy(
                        rows[(j - 1) % 2], agg_sh.at[dv.at[j - 1]],
                        semsc[(j - 1) % 2]).wait()
                    pltpu.make_async_copy(
                        ones_v, deg_sh.at[dv.at[j - 1]],
                        semsc[(j - 1) % 2]).wait()
                if j + 1 < SBCH:
                    pltpu.async_copy(
                        xw.at[sv.at[j + 1]], rows[(j + 1) % 2],
                        semg[(j + 1) % 2])
                else:
                    @pl.when(b < NBLK - 1)
                    def _():
                        pltpu.make_async_copy(
                            srcp.at[c, s, b + 1], nsv, semi_s).wait()
                        pltpu.make_async_copy(
                            dstp.at[c, s, b + 1], ndv, semi_d).wait()
                        pltpu.async_copy(xw.at[nsv.at[0]], rows0, semg0)
                pltpu.async_copy(rows[j % 2], agg_sh.at[dv.at[j]],
                                 semsc[j % 2], add=True)
                pltpu.async_copy(ones_v, deg_sh.at[dv.at[j]],
                                 semsc[j % 2], add=True)

            pltpu.make_async_copy(
                rows[(SBCH - 1) % 2], agg_sh.at[dv.at[SBCH - 1]],
                semsc[(SBCH - 1) % 2]).wait()
            pltpu.make_async_copy(
                ones_v, deg_sh.at[dv.at[SBCH - 1]],
                semsc[(SBCH - 1) % 2]).wait()

            @pl.when(b < NBLK - 2)
            def _():
                pltpu.async_copy(srcp.at[c, s, b + 2], sv, semi_s)
                pltpu.async_copy(dstp.at[c, s, b + 2], dv, semi_d)
        return _
    lax.fori_loop(0, NBLK // 2, _pair, None)

    plsc.subcore_barrier()

    for t in range(ROWS_W // CHW):
        pltpu.sync_copy(agg_sh.at[pl.ds(s * ROWS_W + t * CHW, CHW)], rows0)
        pltpu.sync_copy(rows0, agg_out.at[c, pl.ds(s * ROWS_W + t * CHW, CHW)])
    pltpu.sync_copy(deg_sh.at[pl.ds(s * ROWS_W, ROWS_W)], zvec)
    pltpu.sync_copy(zvec, deg_out.at[c, pl.ds(s * ROWS_W, ROWS_W)])


_edge_call = functools.partial(
    pl.kernel,
    _edge_body,
    out_type=(
        jax.ShapeDtypeStruct((2, NPAD, D), jnp.float32),
        jax.ShapeDtypeStruct((2, NPAD), jnp.float32),
    ),
    mesh=_sc_mesh,
    scratch_types=[
        pltpu.VMEM_SHARED((NPAD, D), jnp.float32),
        pltpu.VMEM_SHARED((NPAD,), jnp.float32),
        pltpu.VMEM((SBCH, CHW), jnp.int32),
        pltpu.VMEM((SBCH, CHW), jnp.int32),
        pltpu.VMEM((SBCH, CHW), jnp.int32),
        pltpu.VMEM((SBCH, CHW), jnp.int32),
        pltpu.VMEM((CHW, D), jnp.float32),
        pltpu.VMEM((CHW, D), jnp.float32),
        pltpu.VMEM((ROWS_W,), jnp.float32),
        pltpu.VMEM((CHW,), jnp.float32),
        pltpu.SemaphoreType.DMA,
        pltpu.SemaphoreType.DMA,
        pltpu.SemaphoreType.DMA,
        pltpu.SemaphoreType.DMA,
        pltpu.SemaphoreType.DMA,
        pltpu.SemaphoreType.DMA,
    ],
)


def _post_body(agg_ref, deg_ref, xr_ref, w1_ref, bs1_ref, w2_ref,
               o_ref, h_sc, s_sc):
    t = pl.program_id(0)

    @pl.when(t < 20)
    def _pass1():
        i = t % 10
        deg = jnp.maximum(jnp.transpose(deg_ref[0], (1, 0)), 1.0)
        h = jnp.maximum(agg_ref[0] / deg + xr_ref[0], 0.0)
        tt = jnp.tanh(
            lax.dot_general(h, w1_ref[...], (((1,), (1,)), ((), ())),
                            preferred_element_type=jnp.float32)
            + bs1_ref[...][None, :]
        )
        rows = i * 1024 + lax.broadcasted_iota(jnp.int32, (1024, 1), 0)
        maskf = (rows < NREG).astype(jnp.float32)
        contrib = jnp.sum(tt * w2_ref[...][0][None, :] * maskf)
        base = pl.multiple_of(i * 1024, 1024)

        @pl.when(t < 10)
        def _():
            h_sc[0, pl.ds(base, 1024)] = h
            s_sc[0] = jnp.where(i == 0, 0.0, s_sc[0]) + contrib

        @pl.when(t >= 10)
        def _():
            h_sc[1, pl.ds(base, 1024)] = h
            s_sc[1] = jnp.where(i == 0, 0.0, s_sc[1]) + contrib

    @pl.when(t >= 20)
    def _pass2():
        i = t - 20
        base = pl.multiple_of(i * 1000, 1000)
        s0 = s_sc[0] / float(NREG)
        s1 = s_sc[1] / float(NREG)
        m = jnp.maximum(s0, s1)
        e0 = jnp.exp(s0 - m)
        e1 = jnp.exp(s1 - m)
        b0 = e0 / (e0 + e1)
        o_ref[...] = (b0 * h_sc[0, pl.ds(base, 1000)]
                      + (1.0 - b0) * h_sc[1, pl.ds(base, 1000)])


def _post_call(agg, deg, xr, w1, bs1, w2):
    return pl.pallas_call(
        _post_body,
        grid=(30,),
        in_specs=[
            pl.BlockSpec((1, 1024, D),
                         lambda t: (jnp.where(t < 20, t // 10, 0),
                                    jnp.where(t < 20, t % 10, 0), 0)),
            pl.BlockSpec((1, 1, 1024),
                         lambda t: (jnp.where(t < 20, t, 0), 0, 0)),
            pl.BlockSpec((1, 1024, D),
                         lambda t: (jnp.where(t < 20, t // 10, 0),
                                    jnp.where(t < 20, t % 10, 0), 0)),
            pl.BlockSpec((HIDDEN, D), lambda t: (0, 0)),
            pl.BlockSpec((HIDDEN,), lambda t: (0,)),
            pl.BlockSpec((1, HIDDEN), lambda t: (0, 0)),
        ],
        out_specs=pl.BlockSpec(
            (1000, D), lambda t: (jnp.where(t >= 20, t - 20, 0), 0)),
        out_shape=jax.ShapeDtypeStruct((NREG, D), jnp.float32),
        scratch_shapes=[pltpu.VMEM((2, NPAD, D), jnp.float32),
                        pltpu.SMEM((2,), jnp.float32)],
    )(agg, deg.reshape(2 * NPAD // 1024, 1, 1024), xr, w1, bs1, w2)


def kernel(E, edge_index0, eids0, edge_index1, eids1, Wrel0, Wroot0, b0,
           Wrel1, Wroot1, b1, W1, bs1, W2, ifdropout):
    del ifdropout
    pad_rows = (jnp.arange(NPAD - N_SUB, dtype=jnp.int32) * 199) % N_TOTAL
    eids = jnp.stack([eids0, eids1])
    eids_pad = jnp.concatenate(
        [eids, jnp.broadcast_to(pad_rows, (2, NPAD - N_SUB))], axis=1
    ).reshape(2, NTILE, GCH, 128)

    padn = EPAD - N_EDGES
    ar = jnp.arange(padn, dtype=jnp.int32)
    pad_dst = N_SUB + (ar % (NPAD - N_SUB))
    src_idx = jnp.concatenate(
        [edge_index0[0], pad_dst, edge_index1[0] + NPAD, NPAD + pad_dst]
    ).reshape(2, NTILE, NBLK, SBCH, CHW)
    dst_idx = jnp.concatenate(
        [edge_index0[1], pad_dst, edge_index1[1], pad_dst]
    ).reshape(2, NTILE, NBLK, SBCH, CHW)

    wrel = jnp.stack([Wrel0[0], Wrel1[0]])
    wroot = jnp.stack([Wroot0, Wroot1])
    bstack = jnp.stack([b0, b1])

    x = _gather_call()(E, eids_pad)
    xw, xr = _mm_call(x, wrel, wroot, bstack)
    agg, deg = _edge_call()(xw.reshape(2 * NPAD, D), src_idx, dst_idx)
    return _post_call(agg, deg, xr, W1, bs1, W2)
